# speedup vs baseline: 1.5874x; 1.5874x over previous
import jax
import jax.numpy as jnp
from jax.experimental import pallas as pl

ETA = 0.5


def _matmul_kernel(z_ref, w_ref, b_ref, o_ref):
    o_ref[...] = jnp.dot(z_ref[...], w_ref[...],
                         preferred_element_type=jnp.float32) + b_ref[...]


def kernel(h, tax, edge_index, wh_w, W_w, W_b):
    n = h.shape[0]
    src = edge_index[0]
    dst = edge_index[1]
    h_src = h[src]
    wf = h[src] @ wh_w[:, :128].T + h[dst] @ wh_w[:, 128:].T
    wf = jax.nn.leaky_relu(wf, negative_slope=0.01)
    e_f = jnp.exp(wf)
    s_f = jax.ops.segment_sum(e_f, dst, num_segments=n)
    alpha_f = e_f / s_f[dst]
    wt = jnp.sum(tax[src] * tax[dst], axis=1, keepdims=True)
    e_t = jnp.exp(wt)
    s_t = jax.ops.segment_sum(e_t, dst, num_segments=n)
    alpha_t = e_t / s_t[dst]
    alpha = ETA * alpha_f + (1.0 - ETA) * alpha_t
    z = jax.ops.segment_sum(h_src * alpha, dst, num_segments=n)
    out = pl.pallas_call(
        _matmul_kernel,
        out_shape=jax.ShapeDtypeStruct((n, W_w.shape[0]), jnp.float32),
    )(z, W_w.T, W_b[None, :])
    return out


# speedup vs baseline: 15.6546x; 9.8618x over previous
import dataclasses
import functools

import jax
import jax.numpy as jnp
from jax import lax
from jax.experimental import pallas as pl
from jax.experimental.pallas import tpu as pltpu
from jax.experimental.pallas import tpu_sc as plsc

N = 10000
E = 320000
D = 128
ETA = 0.5

NC = 2
NS = 16
NW = NC * NS
L = 16
CHUNK = 128
NBLK = E // CHUNK
TILE_ITERS = (NBLK + NW - 1) // NW

TROW = 80

ZBLK = 16
NZB = N // ZBLK
ZB_ITERS = (NZB + NS - 1) // NS
FBLK = 128
NFB = N // FBLK
FB_ITERS = 5

_mesh = plsc.VectorSubcoreMesh(core_axis_name="c", subcore_axis_name="s")

_sc_params = pltpu.CompilerParams()
if "needs_layout_passes" in pltpu.CompilerParams.__dataclass_fields__:
    _sc_params = dataclasses.replace(_sc_params, needs_layout_passes=False)

_DNUMS = lax.GatherDimensionNumbers(
    offset_dims=(), collapsed_slice_dims=(0,), start_index_map=(0,))


def _dg(v, idx):
    return lax.gather(v, idx[:, None], _DNUMS, (1,),
                      mode=lax.GatherScatterMode.PROMISE_IN_BOUNDS)


def _rc(k):
    return lax.shift_right_logical(k, 7), lax.bitwise_and(k, D - 1)


def _ab_body(h3_ref, w1_ref, w2_ref, a_ref, b_ref):
    hm = h3_ref[...]
    a_ref[...] = jnp.sum(hm * w1_ref[...][0][None, None, :], axis=2)
    b_ref[...] = jnp.sum(hm * w2_ref[...][0][None, None, :], axis=2)


def _q_body(mtp_ref, stp_ref, sfp_ref, qf_ref, qt_ref):
    sf_tot = jnp.sum(sfp_ref[...], axis=0)
    m = jnp.max(mtp_ref[...], axis=0)
    st_tot = jnp.sum(stp_ref[...] * jnp.exp(mtp_ref[...] - m[None]), axis=0)
    qf_ref[...] = jnp.log(sf_tot)
    qt_ref[...] = m + jnp.log(st_tot)


def _out_body(z_ref, w_ref, b_ref, o_ref):
    zc = z_ref[0] + z_ref[1]
    o_ref[...] = lax.dot_general(
        zc, w_ref[...], (((1,), (1,)), ((), ())),
        preferred_element_type=jnp.float32) + b_ref[...]


@functools.partial(
    pl.kernel,
    out_type=[
        jax.ShapeDtypeStruct((E,), jnp.float32),
        jax.ShapeDtypeStruct((E,), jnp.float32),
        jax.ShapeDtypeStruct((NW, TROW, D), jnp.float32),
        jax.ShapeDtypeStruct((NW, TROW, D), jnp.float32),
        jax.ShapeDtypeStruct((NW, TROW, D), jnp.float32),
    ],
    mesh=_mesh,
    compiler_params=_sc_params,
    scratch_types=[
        pltpu.VMEM((CHUNK,), jnp.int32),
        pltpu.VMEM((CHUNK,), jnp.int32),
        pltpu.VMEM((CHUNK, D), jnp.float32),
        pltpu.VMEM((CHUNK, D), jnp.float32),
        pltpu.VMEM((CHUNK,), jnp.float32),
        pltpu.VMEM((CHUNK,), jnp.float32),
        pltpu.VMEM((TROW, D), jnp.float32),
        pltpu.VMEM((TROW, D), jnp.float32),
        pltpu.VMEM((TROW, D), jnp.float32),
        pltpu.VMEM((TROW, D), jnp.float32),
        pltpu.VMEM((TROW, D), jnp.float32),
        pltpu.SemaphoreType.DMA,
        pltpu.SemaphoreType.DMA,
    ],
)
def _pass_a(tax_hbm, ei_hbm, a_hbm, b_hbm, neg_hbm, zero_hbm,
            wf_hbm, wt_hbm, mtp_hbm, stp_hbm, sfp_hbm,
            sidx, didx, ts, td, wfv, wtv, at, bt, mtw, stw, sfw,
            sem1, sem2):
    cid = lax.axis_index("c")
    sid = lax.axis_index("s")
    wid = sid * NC + cid
    iota = lax.iota(jnp.int32, L)
    lastlane = iota == (L - 1)
    idxp1 = jnp.minimum(iota + 1, L - 1)
    ones = jnp.full((L,), 1.0, jnp.float32)

    pltpu.sync_copy(a_hbm, at)
    pltpu.sync_copy(b_hbm, bt)
    pltpu.sync_copy(neg_hbm, mtw)
    pltpu.sync_copy(zero_hbm, stw)
    pltpu.sync_copy(zero_hbm, sfw)

    @pl.loop(0, TILE_ITERS)
    def _(t):
        c = wid + t * NW

        @pl.when(c < NBLK)
        def _():
            base = c * CHUNK
            pltpu.sync_copy(ei_hbm.at[0, pl.ds(base, CHUNK)], sidx)
            pltpu.sync_copy(ei_hbm.at[1, pl.ds(base, CHUNK)], didx)
            cp1 = pltpu.async_copy(tax_hbm.at[sidx], ts, sem1)
            cp2 = pltpu.async_copy(tax_hbm.at[didx], td, sem2)
            cp1.wait()
            cp2.wait()

            @pl.loop(0, CHUNK)
            def _(e):
                acc = ts[e, pl.ds(0, L)] * td[e, pl.ds(0, L)]
                for k in range(1, D // L):
                    acc = acc + ts[e, pl.ds(k * L, L)] * td[e, pl.ds(k * L, L)]
                cs = plsc.cumsum(acc)
                plsc.store_scatter(wtv, [jnp.full((L,), e, jnp.int32)], cs,
                                   mask=lastlane)

            @pl.loop(0, CHUNK // L)
            def _(g):
                sv = sidx[pl.ds(g * L, L)]
                dv = didx[pl.ds(g * L, L)]
                sr, sc_ = _rc(sv)
                dr, dc = _rc(dv)
                wf = (plsc.load_gather(at, [sr, sc_])
                      + plsc.load_gather(bt, [dr, dc]))
                wfv[pl.ds(g * L, L)] = jnp.maximum(wf, wf * 0.01)

            pltpu.sync_copy(wfv, wf_hbm.at[pl.ds(base, CHUNK)])
            pltpu.sync_copy(wtv, wt_hbm.at[pl.ds(base, CHUNK)])

            @pl.loop(0, CHUNK // L)
            def _(g):
                dv = didx[pl.ds(g * L, L)]
                kd, perm = plsc.sort_key_val(dv, iota)
                wt_s = _dg(wtv[pl.ds(g * L, L)], perm)
                wf_s = _dg(wfv[pl.ds(g * L, L)], perm)

                knext = _dg(kd, idxp1)
                is_last = (knext != kd) | lastlane

                v = jnp.exp(wf_s)
                mval = wt_s
                sval = ones
                for s in (1, 2, 4, 8):
                    idxm = jnp.maximum(iota - s, 0)
                    okm = (_dg(kd, idxm) == kd) & (iota >= s)
                    v = v + jnp.where(okm, _dg(v, idxm), 0.0)
                    m1 = _dg(mval, idxm)
                    s1 = _dg(sval, idxm)
                    mm = jnp.maximum(mval, m1)
                    s2 = sval * jnp.exp(mval - mm) + s1 * jnp.exp(m1 - mm)
                    mval = jnp.where(okm, mm, mval)
                    sval = jnp.where(okm, s2, sval)

                kr, kc = _rc(kd)
                cur_f = plsc.load_gather(sfw, [kr, kc])
                plsc.store_scatter(sfw, [kr, kc], cur_f + v, mask=is_last)

                m_cur = plsc.load_gather(mtw, [kr, kc])
                s_cur = plsc.load_gather(stw, [kr, kc])
                mm2 = jnp.maximum(m_cur, mval)
                s_new = (s_cur * jnp.exp(m_cur - mm2)
                         + sval * jnp.exp(mval - mm2))
                plsc.store_scatter(mtw, [kr, kc], mm2, mask=is_last)
                plsc.store_scatter(stw, [kr, kc], s_new, mask=is_last)

    pltpu.sync_copy(mtw, mtp_hbm.at[wid])
    pltpu.sync_copy(stw, stp_hbm.at[wid])
    pltpu.sync_copy(sfw, sfp_hbm.at[wid])


@functools.partial(
    pl.kernel,
    out_type=jax.ShapeDtypeStruct((NC, N, D), jnp.float32),
    mesh=_mesh,
    compiler_params=_sc_params,
    scratch_types=[
        pltpu.VMEM((CHUNK,), jnp.int32),
        pltpu.VMEM((CHUNK,), jnp.int32),
        pltpu.VMEM((CHUNK, D), jnp.float32),
        pltpu.VMEM((CHUNK,), jnp.float32),
        pltpu.VMEM((CHUNK,), jnp.float32),
        pltpu.VMEM((CHUNK,), jnp.float32),
        pltpu.VMEM((TROW, D), jnp.float32),
        pltpu.VMEM((TROW, D), jnp.float32),
        pltpu.VMEM((ZBLK, D), jnp.float32),
        pltpu.VMEM((ZBLK,), jnp.int32),
        pltpu.VMEM((FBLK,), jnp.int32),
        pltpu.VMEM_SHARED((N, D), jnp.float32),
        pltpu.SemaphoreType.DMA,
    ],
)
def _pass_c(h_hbm, ei_hbm, wf_hbm, wt_hbm, qf_hbm, qt_hbm, zero_hbm, z_hbm,
            sidx, didx, hs, wfv, wtv, alphab, qft, qtt, zb16, zidx, fidx,
            z_sh, sem1):
    cid = lax.axis_index("c")
    sid = lax.axis_index("s")
    wid = sid * NC + cid
    iota = lax.iota(jnp.int32, L)

    pltpu.sync_copy(qf_hbm, qft)
    pltpu.sync_copy(qt_hbm, qtt)
    pltpu.sync_copy(zero_hbm, zb16)

    @pl.loop(0, ZB_ITERS)
    def _(k):
        blk = sid + k * NS

        @pl.when(blk < NZB)
        def _():
            zidx[...] = blk * ZBLK + iota
            pltpu.sync_copy(zb16, z_sh.at[zidx])

    plsc.subcore_barrier()

    @pl.loop(0, TILE_ITERS)
    def _(t):
        c = wid + t * NW

        @pl.when(c < NBLK)
        def _():
            base = c * CHUNK
            pltpu.sync_copy(ei_hbm.at[0, pl.ds(base, CHUNK)], sidx)
            pltpu.sync_copy(ei_hbm.at[1, pl.ds(base, CHUNK)], didx)
            cp1 = pltpu.async_copy(h_hbm.at[sidx], hs, sem1)
            pltpu.sync_copy(wf_hbm.at[pl.ds(base, CHUNK)], wfv)
            pltpu.sync_copy(wt_hbm.at[pl.ds(base, CHUNK)], wtv)

            @pl.loop(0, CHUNK // L)
            def _(g):
                dv = didx[pl.ds(g * L, L)]
                dr, dc = _rc(dv)
                qf = plsc.load_gather(qft, [dr, dc])
                qt = plsc.load_gather(qtt, [dr, dc])
                af = jnp.exp(wfv[pl.ds(g * L, L)] - qf)
                at_ = jnp.exp(wtv[pl.ds(g * L, L)] - qt)
                alphab[pl.ds(g * L, L)] = ETA * af + (1.0 - ETA) * at_

            cp1.wait()

            @pl.loop(0, CHUNK)
            def _(e):
                av = plsc.load_gather(alphab, [jnp.full((L,), e, jnp.int32)])
                for k in range(D // L):
                    hs[e, pl.ds(k * L, L)] = hs[e, pl.ds(k * L, L)] * av

            pltpu.sync_copy(hs, z_sh.at[didx], add=True)

    plsc.subcore_barrier()

    @pl.loop(0, FB_ITERS)
    def _(k):
        blk = sid + k * NS

        @pl.when(blk < NFB)
        def _():
            @pl.loop(0, FBLK // L)
            def _(g):
                fidx[pl.ds(g * L, L)] = blk * FBLK + g * L + iota

            pltpu.sync_copy(z_sh.at[fidx], hs)
            pltpu.sync_copy(hs, z_hbm.at[cid, pl.ds(blk * FBLK, FBLK)])

        @pl.when(blk == NFB)
        def _():
            zidx[...] = NFB * FBLK + iota
            pltpu.sync_copy(z_sh.at[zidx], zb16)
            pltpu.sync_copy(zb16, z_hbm.at[cid, pl.ds(NFB * FBLK, ZBLK)])


def kernel(h, tax, edge_index, wh_w, W_w, W_b):
    w1 = wh_w[:, :D]
    w2 = wh_w[:, D:]
    h3 = jnp.concatenate(
        [h, jnp.zeros((TROW * D - N, D), jnp.float32)]
    ).reshape(TROW, D, D)

    a2, b2 = pl.pallas_call(
        _ab_body,
        out_shape=[jax.ShapeDtypeStruct((TROW, D), jnp.float32),
                   jax.ShapeDtypeStruct((TROW, D), jnp.float32)],
    )(h3, w1, w2)

    neg = jnp.full((TROW, D), -1e30, jnp.float32)
    zero = jnp.zeros((TROW, D), jnp.float32)

    wf, wt, mtp, stp, sfp = _pass_a(tax, edge_index, a2, b2, neg, zero)

    qf, qt = pl.pallas_call(
        _q_body,
        out_shape=[jax.ShapeDtypeStruct((TROW, D), jnp.float32),
                   jax.ShapeDtypeStruct((TROW, D), jnp.float32)],
    )(mtp, stp, sfp)

    zero16 = jnp.zeros((ZBLK, D), jnp.float32)
    z = _pass_c(h, edge_index, wf, wt, qf, qt, zero16)

    out = pl.pallas_call(
        _out_body,
        out_shape=jax.ShapeDtypeStruct((N, D), jnp.float32),
    )(z, W_w, W_b[None, :])
    return out


# speedup vs baseline: 18.7313x; 1.1965x over previous
import dataclasses
import functools

import jax
import jax.numpy as jnp
from jax import lax
from jax.experimental import pallas as pl
from jax.experimental.pallas import tpu as pltpu
from jax.experimental.pallas import tpu_sc as plsc

N = 10000
E = 320000
D = 128
ETA = 0.5

NC = 2
NS = 16
NW = NC * NS
L = 16
CHUNK = 128
NBLK = E // CHUNK
TILE_ITERS = (NBLK + NW - 1) // NW

TROW = 80

ZBLK = 16
NZB = N // ZBLK
ZB_ITERS = (NZB + NS - 1) // NS
FBLK = 128
NFB = N // FBLK
FB_ITERS = 5

_mesh = plsc.VectorSubcoreMesh(core_axis_name="c", subcore_axis_name="s")

_sc_params = pltpu.CompilerParams()
if "needs_layout_passes" in pltpu.CompilerParams.__dataclass_fields__:
    _sc_params = dataclasses.replace(_sc_params, needs_layout_passes=False)

_DNUMS = lax.GatherDimensionNumbers(
    offset_dims=(), collapsed_slice_dims=(0,), start_index_map=(0,))


def _dg(v, idx):
    return lax.gather(v, idx[:, None], _DNUMS, (1,),
                      mode=lax.GatherScatterMode.PROMISE_IN_BOUNDS)


def _rc(k):
    return lax.shift_right_logical(k, 7), lax.bitwise_and(k, D - 1)


def _ab_body(h3_ref, w1_ref, w2_ref, a_ref, b_ref):
    hm = h3_ref[...]
    a_ref[...] = jnp.sum(hm * w1_ref[...][0][None, None, :], axis=2)
    b_ref[...] = jnp.sum(hm * w2_ref[...][0][None, None, :], axis=2)


def _q_body(mtp_ref, stp_ref, sfp_ref, qf_ref, qt_ref):
    sf_tot = jnp.sum(sfp_ref[...], axis=0)
    m = jnp.max(mtp_ref[...], axis=0)
    st_tot = jnp.sum(stp_ref[...] * jnp.exp(mtp_ref[...] - m[None]), axis=0)
    qf_ref[...] = jnp.log(sf_tot)
    qt_ref[...] = m + jnp.log(st_tot)


def _out_body(z_ref, w_ref, b_ref, o_ref):
    zc = z_ref[0] + z_ref[1]
    o_ref[...] = lax.dot_general(
        zc, w_ref[...], (((1,), (1,)), ((), ())),
        preferred_element_type=jnp.float32) + b_ref[...]


@functools.partial(
    pl.kernel,
    out_type=[
        jax.ShapeDtypeStruct((E,), jnp.float32),
        jax.ShapeDtypeStruct((E,), jnp.float32),
        jax.ShapeDtypeStruct((NW, TROW, D), jnp.float32),
        jax.ShapeDtypeStruct((NW, TROW, D), jnp.float32),
        jax.ShapeDtypeStruct((NW, TROW, D), jnp.float32),
    ],
    mesh=_mesh,
    compiler_params=_sc_params,
    scratch_types=[
        pltpu.VMEM((CHUNK,), jnp.int32),
        pltpu.VMEM((CHUNK,), jnp.int32),
        pltpu.VMEM((CHUNK, D), jnp.float32),
        pltpu.VMEM((CHUNK, D), jnp.float32),
        pltpu.VMEM((CHUNK,), jnp.int32),
        pltpu.VMEM((CHUNK,), jnp.int32),
        pltpu.VMEM((CHUNK, D), jnp.float32),
        pltpu.VMEM((CHUNK, D), jnp.float32),
        pltpu.VMEM((CHUNK,), jnp.float32),
        pltpu.VMEM((CHUNK,), jnp.float32),
        pltpu.VMEM((TROW, D), jnp.float32),
        pltpu.VMEM((TROW, D), jnp.float32),
        pltpu.VMEM((TROW, D), jnp.float32),
        pltpu.VMEM((TROW, D), jnp.float32),
        pltpu.VMEM((TROW, D), jnp.float32),
        pltpu.SemaphoreType.DMA,
        pltpu.SemaphoreType.DMA,
        pltpu.SemaphoreType.DMA,
        pltpu.SemaphoreType.DMA,
    ],
)
def _pass_a(tax_hbm, ei_hbm, a_hbm, b_hbm, neg_hbm, zero_hbm,
            wf_hbm, wt_hbm, mtp_hbm, stp_hbm, sfp_hbm,
            sidx0, didx0, ts0, td0, sidx1, didx1, ts1, td1,
            wfv, wtv, at, bt, mtw, stw, sfw,
            sem1, sem2, sem3, sem4):
    cid = lax.axis_index("c")
    sid = lax.axis_index("s")
    wid = sid * NC + cid
    iota = lax.iota(jnp.int32, L)
    lastlane = iota == (L - 1)
    idxp1 = jnp.minimum(iota + 1, L - 1)
    ones = jnp.full((L,), 1.0, jnp.float32)

    pltpu.sync_copy(a_hbm, at)
    pltpu.sync_copy(b_hbm, bt)
    pltpu.sync_copy(neg_hbm, mtw)
    pltpu.sync_copy(zero_hbm, stw)
    pltpu.sync_copy(zero_hbm, sfw)

    def issue(c, sidx, didx, ts, td, semA, semB):
        base = c * CHUNK
        pltpu.sync_copy(ei_hbm.at[0, pl.ds(base, CHUNK)], sidx)
        pltpu.sync_copy(ei_hbm.at[1, pl.ds(base, CHUNK)], didx)
        pltpu.async_copy(tax_hbm.at[sidx], ts, semA)
        pltpu.async_copy(tax_hbm.at[didx], td, semB)

    def compute(c, sidx, didx, ts, td, semA, semB):
        base = c * CHUNK
        pltpu.make_async_copy(tax_hbm.at[sidx], ts, semA).wait()
        pltpu.make_async_copy(tax_hbm.at[didx], td, semB).wait()

        @pl.loop(0, CHUNK)
        def _(e):
            acc = ts[e, pl.ds(0, L)] * td[e, pl.ds(0, L)]
            for k in range(1, D // L):
                acc = acc + ts[e, pl.ds(k * L, L)] * td[e, pl.ds(k * L, L)]
            cs = plsc.cumsum(acc)
            plsc.store_scatter(wtv, [jnp.full((L,), e, jnp.int32)], cs,
                               mask=lastlane)

        @pl.loop(0, CHUNK // L)
        def _(g):
            sv = sidx[pl.ds(g * L, L)]
            dv = didx[pl.ds(g * L, L)]
            sr, sc_ = _rc(sv)
            dr, dc = _rc(dv)
            wf = (plsc.load_gather(at, [sr, sc_])
                  + plsc.load_gather(bt, [dr, dc]))
            wfv[pl.ds(g * L, L)] = jnp.maximum(wf, wf * 0.01)

        pltpu.sync_copy(wfv, wf_hbm.at[pl.ds(base, CHUNK)])
        pltpu.sync_copy(wtv, wt_hbm.at[pl.ds(base, CHUNK)])

        @pl.loop(0, CHUNK // L)
        def _(g):
            dv = didx[pl.ds(g * L, L)]
            kd, perm = plsc.sort_key_val(dv, iota)
            wt_s = _dg(wtv[pl.ds(g * L, L)], perm)
            wf_s = _dg(wfv[pl.ds(g * L, L)], perm)

            knext = _dg(kd, idxp1)
            is_last = (knext != kd) | lastlane

            v = jnp.exp(wf_s)
            mval = wt_s
            sval = ones
            for s in (1, 2, 4, 8):
                idxm = jnp.maximum(iota - s, 0)
                okm = (_dg(kd, idxm) == kd) & (iota >= s)
                v = v + jnp.where(okm, _dg(v, idxm), 0.0)
                m1 = _dg(mval, idxm)
                s1 = _dg(sval, idxm)
                mm = jnp.maximum(mval, m1)
                s2 = sval * jnp.exp(mval - mm) + s1 * jnp.exp(m1 - mm)
                mval = jnp.where(okm, mm, mval)
                sval = jnp.where(okm, s2, sval)

            kr, kc = _rc(kd)
            cur_f = plsc.load_gather(sfw, [kr, kc])
            plsc.store_scatter(sfw, [kr, kc], cur_f + v, mask=is_last)

            m_cur = plsc.load_gather(mtw, [kr, kc])
            s_cur = plsc.load_gather(stw, [kr, kc])
            mm2 = jnp.maximum(m_cur, mval)
            s_new = (s_cur * jnp.exp(m_cur - mm2)
                     + sval * jnp.exp(mval - mm2))
            plsc.store_scatter(mtw, [kr, kc], mm2, mask=is_last)
            plsc.store_scatter(stw, [kr, kc], s_new, mask=is_last)

    @pl.when(wid < NBLK)
    def _():
        issue(wid, sidx0, didx0, ts0, td0, sem1, sem2)

    @pl.loop(0, (TILE_ITERS + 1) // 2)
    def _(u):
        ce = wid + (2 * u) * NW
        co = ce + NW

        @pl.when(co < NBLK)
        def _():
            issue(co, sidx1, didx1, ts1, td1, sem3, sem4)

        @pl.when(ce < NBLK)
        def _():
            compute(ce, sidx0, didx0, ts0, td0, sem1, sem2)

        @pl.when(ce + 2 * NW < NBLK)
        def _():
            issue(ce + 2 * NW, sidx0, didx0, ts0, td0, sem1, sem2)

        @pl.when(co < NBLK)
        def _():
            compute(co, sidx1, didx1, ts1, td1, sem3, sem4)

    pltpu.sync_copy(mtw, mtp_hbm.at[wid])
    pltpu.sync_copy(stw, stp_hbm.at[wid])
    pltpu.sync_copy(sfw, sfp_hbm.at[wid])


@functools.partial(
    pl.kernel,
    out_type=jax.ShapeDtypeStruct((NC, N, D), jnp.float32),
    mesh=_mesh,
    compiler_params=_sc_params,
    scratch_types=[
        pltpu.VMEM((CHUNK,), jnp.int32),
        pltpu.VMEM((CHUNK,), jnp.int32),
        pltpu.VMEM((CHUNK, D), jnp.float32),
        pltpu.VMEM((CHUNK,), jnp.float32),
        pltpu.VMEM((CHUNK,), jnp.float32),
        pltpu.VMEM((CHUNK,), jnp.float32),
        pltpu.VMEM((TROW, D), jnp.float32),
        pltpu.VMEM((TROW, D), jnp.float32),
        pltpu.VMEM((ZBLK, D), jnp.float32),
        pltpu.VMEM((ZBLK,), jnp.int32),
        pltpu.VMEM((FBLK,), jnp.int32),
        pltpu.VMEM_SHARED((N, D), jnp.float32),
        pltpu.SemaphoreType.DMA,
    ],
)
def _pass_c(h_hbm, ei_hbm, wf_hbm, wt_hbm, qf_hbm, qt_hbm, zero_hbm, z_hbm,
            sidx, didx, hs, wfv, wtv, alphab, qft, qtt, zb16, zidx, fidx,
            z_sh, sem1):
    cid = lax.axis_index("c")
    sid = lax.axis_index("s")
    wid = sid * NC + cid
    iota = lax.iota(jnp.int32, L)

    pltpu.sync_copy(qf_hbm, qft)
    pltpu.sync_copy(qt_hbm, qtt)
    pltpu.sync_copy(zero_hbm, zb16)

    @pl.loop(0, ZB_ITERS)
    def _(k):
        blk = sid + k * NS

        @pl.when(blk < NZB)
        def _():
            zidx[...] = blk * ZBLK + iota
            pltpu.sync_copy(zb16, z_sh.at[zidx])

    plsc.subcore_barrier()

    @pl.loop(0, TILE_ITERS)
    def _(t):
        c = wid + t * NW

        @pl.when(c < NBLK)
        def _():
            base = c * CHUNK
            pltpu.sync_copy(ei_hbm.at[0, pl.ds(base, CHUNK)], sidx)
            pltpu.sync_copy(ei_hbm.at[1, pl.ds(base, CHUNK)], didx)
            cp1 = pltpu.async_copy(h_hbm.at[sidx], hs, sem1)
            pltpu.sync_copy(wf_hbm.at[pl.ds(base, CHUNK)], wfv)
            pltpu.sync_copy(wt_hbm.at[pl.ds(base, CHUNK)], wtv)

            @pl.loop(0, CHUNK // L)
            def _(g):
                dv = didx[pl.ds(g * L, L)]
                dr, dc = _rc(dv)
                qf = plsc.load_gather(qft, [dr, dc])
                qt = plsc.load_gather(qtt, [dr, dc])
                af = jnp.exp(wfv[pl.ds(g * L, L)] - qf)
                at_ = jnp.exp(wtv[pl.ds(g * L, L)] - qt)
                alphab[pl.ds(g * L, L)] = ETA * af + (1.0 - ETA) * at_

            cp1.wait()

            @pl.loop(0, CHUNK)
            def _(e):
                av = plsc.load_gather(alphab, [jnp.full((L,), e, jnp.int32)])
                for k in range(D // L):
                    hs[e, pl.ds(k * L, L)] = hs[e, pl.ds(k * L, L)] * av

            pltpu.sync_copy(hs, z_sh.at[didx], add=True)

    plsc.subcore_barrier()

    @pl.loop(0, FB_ITERS)
    def _(k):
        blk = sid + k * NS

        @pl.when(blk < NFB)
        def _():
            @pl.loop(0, FBLK // L)
            def _(g):
                fidx[pl.ds(g * L, L)] = blk * FBLK + g * L + iota

            pltpu.sync_copy(z_sh.at[fidx], hs)
            pltpu.sync_copy(hs, z_hbm.at[cid, pl.ds(blk * FBLK, FBLK)])

        @pl.when(blk == NFB)
        def _():
            zidx[...] = NFB * FBLK + iota
            pltpu.sync_copy(z_sh.at[zidx], zb16)
            pltpu.sync_copy(zb16, z_hbm.at[cid, pl.ds(NFB * FBLK, ZBLK)])


def kernel(h, tax, edge_index, wh_w, W_w, W_b):
    w1 = wh_w[:, :D]
    w2 = wh_w[:, D:]
    h3 = jnp.concatenate(
        [h, jnp.zeros((TROW * D - N, D), jnp.float32)]
    ).reshape(TROW, D, D)

    a2, b2 = pl.pallas_call(
        _ab_body,
        out_shape=[jax.ShapeDtypeStruct((TROW, D), jnp.float32),
                   jax.ShapeDtypeStruct((TROW, D), jnp.float32)],
    )(h3, w1, w2)

    neg = jnp.full((TROW, D), -1e30, jnp.float32)
    zero = jnp.zeros((TROW, D), jnp.float32)

    wf, wt, mtp, stp, sfp = _pass_a(tax, edge_index, a2, b2, neg, zero)

    qf, qt = pl.pallas_call(
        _q_body,
        out_shape=[jax.ShapeDtypeStruct((TROW, D), jnp.float32),
                   jax.ShapeDtypeStruct((TROW, D), jnp.float32)],
    )(mtp, stp, sfp)

    zero16 = jnp.zeros((ZBLK, D), jnp.float32)
    z = _pass_c(h, edge_index, wf, wt, qf, qt, zero16)

    out = pl.pallas_call(
        _out_body,
        out_shape=jax.ShapeDtypeStruct((N, D), jnp.float32),
    )(z, W_w, W_b[None, :])
    return out
